# baseline (speedup 1.0000x reference)
import jax
import jax.numpy as jnp
from jax.experimental import pallas as pl

BEAT_LEN = 128
TOKEN_DIM = 64


def _proj_kernel(x_ref, wt_ref, b_ref, o_ref, bi_ref):
    BL = x_ref.shape[0] * x_ref.shape[1]
    N = x_ref.shape[2] // BEAT_LEN
    x = x_ref[...].reshape(BL * N, BEAT_LEN)
    y = (
        jnp.dot(x, wt_ref[...], preferred_element_type=jnp.float32)
        + b_ref[...]
    )
    o_ref[...] = y.reshape(o_ref.shape)
    bi_ref[...] = jnp.full(bi_ref.shape, float(BEAT_LEN), dtype=jnp.float32)


@jax.jit
def _run(ecg, W, b):
    B, L, T = ecg.shape
    N = T // BEAT_LEN
    wt = W.T
    b2 = b.reshape(1, TOKEN_DIM)

    X, bi = pl.pallas_call(
        _proj_kernel,
        in_specs=[
            pl.BlockSpec((B, L, T), lambda: (0, 0, 0)),
            pl.BlockSpec((BEAT_LEN, TOKEN_DIM), lambda: (0, 0)),
            pl.BlockSpec((1, TOKEN_DIM), lambda: (0, 0)),
        ],
        out_specs=[
            pl.BlockSpec((B, L, N, TOKEN_DIM), lambda: (0, 0, 0, 0)),
            pl.BlockSpec((B, N), lambda: (0, 0)),
        ],
        out_shape=[
            jax.ShapeDtypeStruct((B, L, N, TOKEN_DIM), jnp.float32),
            jax.ShapeDtypeStruct((B, N), jnp.float32),
        ],
    )(ecg, wt, b2)

    return (X, bi)


def kernel(ecg, W, b):
    return _run(ecg, W, b)
